# baseline (speedup 1.0000x reference)
import jax
import jax.numpy as jnp
from jax import lax
from jax.experimental import pallas as pl
from jax.experimental.pallas import tpu as pltpu
from jax.experimental.pallas import tpu_sc as plsc

N, E, D, DE, H, O = 10000, 320000, 128, 16, 128, 128

NC, NS, L = 2, 16, 16
NW = NC * NS
EPW = E // NW
BLK = 40
NBLK = EPW // BLK
NPAD = 10240
RPT = NPAD // NS


def _xw_body(x_ref, w_ref, b_ref, o_ref):
    o_ref[...] = (
        jnp.dot(x_ref[...], w_ref[...], preferred_element_type=jnp.float32)
        + b_ref[...]
    )


def _xw(x, w1x, b1):
    return pl.pallas_call(
        _xw_body,
        grid=(10,),
        in_specs=[
            pl.BlockSpec((1000, D), lambda i: (i, 0)),
            pl.BlockSpec((D, H), lambda i: (0, 0)),
            pl.BlockSpec((1, H), lambda i: (0, 0)),
        ],
        out_specs=pl.BlockSpec((1000, H), lambda i: (i, 0)),
        out_shape=jax.ShapeDtypeStruct((N, H), jnp.float32),
    )(x, w1x, b1)


def _ew_body(a_ref, w_ref, o_ref):
    o_ref[...] = jnp.dot(a_ref[...], w_ref[...], preferred_element_type=jnp.float32)


def _ew(ea, w1e):
    return pl.pallas_call(
        _ew_body,
        grid=(40,),
        in_specs=[
            pl.BlockSpec((8000, DE), lambda i: (i, 0)),
            pl.BlockSpec((DE, H), lambda i: (0, 0)),
        ],
        out_specs=pl.BlockSpec((8000, H), lambda i: (i, 0)),
        out_shape=jax.ShapeDtypeStruct((E, H), jnp.float32),
    )(ea, w1e)


def _sc_body(xw_hbm, ew_hbm, row_hbm, col_hbm, s_out, c_out,
             rowi_v, coli_v, gath_v, ew_v, ones_v, cb_v, s_sh, cnt_sh, *sems):
    sem_row = sems[0:2]
    sem_col = sems[2:4]
    sem_ew = sems[4:6]
    sem_g = sems[6:8]

    cid = lax.axis_index("c")
    sid = lax.axis_index("s")
    wid = sid * NC + cid

    def _issue_loads(t, b):
        off = wid * EPW + t * BLK
        pltpu.async_copy(row_hbm.at[pl.ds(off, BLK)], rowi_v.at[b], sem_row[b])
        pltpu.async_copy(col_hbm.at[pl.ds(off, BLK)], coli_v.at[b], sem_col[b])
        pltpu.async_copy(ew_hbm.at[pl.ds(off, BLK)], ew_v.at[b], sem_ew[b])

    def _wait_row(b):
        pltpu.make_async_copy(row_hbm.at[pl.ds(0, BLK)], rowi_v.at[b],
                              sem_row[b]).wait()

    def _wait_col_ew(b):
        pltpu.make_async_copy(col_hbm.at[pl.ds(0, BLK)], coli_v.at[b],
                              sem_col[b]).wait()
        pltpu.make_async_copy(ew_hbm.at[pl.ds(0, BLK)], ew_v.at[b],
                              sem_ew[b]).wait()

    def _issue_gather(b):
        pltpu.async_copy(xw_hbm.at[rowi_v.at[b]], gath_v.at[b], sem_g[b])

    def _wait_gather(b):
        pltpu.make_async_copy(xw_hbm.at[rowi_v.at[b]], gath_v.at[b],
                              sem_g[b]).wait()

    def _scatters(b):
        pltpu.sync_copy(gath_v.at[b], s_sh.at[coli_v.at[b]], add=True)
        pltpu.sync_copy(ones_v, cnt_sh.at[coli_v.at[b]], add=True)

    @pl.loop(0, BLK)
    def _fillz(i):
        for j in range(H // L):
            gath_v[0, i, pl.ds(j * L, L)] = jnp.zeros((L,), jnp.float32)

    @pl.loop(0, RPT // L)
    def _fillb(i):
        cb_v[pl.ds(i * L, L)] = jnp.zeros((L,), jnp.float32)

    ones_v[pl.ds(0, L)] = jnp.full((L,), 1.0, jnp.float32)
    ones_v[pl.ds(L, L)] = jnp.full((L,), 1.0, jnp.float32)
    ones_v[pl.ds(BLK - L, L)] = jnp.full((L,), 1.0, jnp.float32)

    @pl.loop(0, RPT // BLK)
    def _zero(t):
        pltpu.sync_copy(gath_v.at[0], s_sh.at[pl.ds(sid * RPT + t * BLK, BLK)])

    pltpu.sync_copy(cb_v, cnt_sh.at[pl.ds(sid * RPT, RPT)])

    plsc.subcore_barrier()

    _issue_loads(0, 0)
    _wait_row(0)
    _issue_gather(0)

    @pl.loop(0, NBLK // 2)
    def _step(s):
        for b in (0, 1):
            t = s * 2 + b
            _wait_col_ew(b)

            @pl.when(t + 1 < NBLK)
            def _():
                _issue_loads(t + 1, b ^ 1)

            _wait_gather(b)

            @pl.when(t + 1 < NBLK)
            def _():
                _wait_row(b ^ 1)
                _issue_gather(b ^ 1)

            @pl.loop(0, BLK, unroll=2)
            def _row(i):
                for j in range(H // L):
                    g = gath_v[b, i, pl.ds(j * L, L)]
                    e = ew_v[b, i, pl.ds(j * L, L)]
                    gath_v[b, i, pl.ds(j * L, L)] = jnp.maximum(g + e, 0.0)

            _scatters(b)

    plsc.subcore_barrier()

    obase = cid * NPAD + sid * RPT

    @pl.loop(0, RPT // BLK)
    def _wout(t):
        pltpu.sync_copy(s_sh.at[pl.ds(sid * RPT + t * BLK, BLK)], gath_v.at[0])
        pltpu.sync_copy(gath_v.at[0], s_out.at[pl.ds(obase + t * BLK, BLK)])

    pltpu.sync_copy(cnt_sh.at[pl.ds(sid * RPT, RPT)], cb_v)
    pltpu.sync_copy(cb_v, c_out.at[pl.ds(obase, RPT)])


def _sc_aggregate(xw, ew, row, col):
    mesh = plsc.VectorSubcoreMesh(core_axis_name="c", subcore_axis_name="s")
    return pl.kernel(
        _sc_body,
        out_type=(
            jax.ShapeDtypeStruct((NC * NPAD, H), jnp.float32),
            jax.ShapeDtypeStruct((NC * NPAD,), jnp.float32),
        ),
        mesh=mesh,
        scratch_types=[
            pltpu.VMEM((2, BLK), jnp.int32),
            pltpu.VMEM((2, BLK), jnp.int32),
            pltpu.VMEM((2, BLK, H), jnp.float32),
            pltpu.VMEM((2, BLK, H), jnp.float32),
            pltpu.VMEM((BLK,), jnp.float32),
            pltpu.VMEM((RPT,), jnp.float32),
            pltpu.VMEM_SHARED((NPAD, H), jnp.float32),
            pltpu.VMEM_SHARED((NPAD,), jnp.float32),
        ] + [pltpu.SemaphoreType.DMA] * 8,
    )(xw, ew, row, col)


def _out_body(s_ref, c_ref, w_ref, b_ref, o_ref):
    s = s_ref[0] + s_ref[1]
    cnt = (c_ref[0] + c_ref[1])[:, None]
    m = jnp.dot(s, w_ref[...], preferred_element_type=jnp.float32)
    o_ref[...] = m / jnp.maximum(cnt, 1.0) + b_ref[...] * (cnt > 0.0)


def _final(s_parts, c_parts, w2, b2):
    return pl.pallas_call(
        _out_body,
        grid=(10,),
        in_specs=[
            pl.BlockSpec((NC, 1024, H), lambda i: (0, i, 0)),
            pl.BlockSpec((NC, 1024), lambda i: (0, i)),
            pl.BlockSpec((H, O), lambda i: (0, 0)),
            pl.BlockSpec((1, O), lambda i: (0, 0)),
        ],
        out_specs=pl.BlockSpec((1024, O), lambda i: (i, 0)),
        out_shape=jax.ShapeDtypeStruct((NPAD, O), jnp.float32),
    )(s_parts, c_parts, w2, b2)


def kernel(x, edge_index, edge_attr, W1, b1, W2, b2):
    row = edge_index[0]
    col = edge_index[1]
    xw = _xw(x, W1[:D], b1.reshape(1, H))
    ew = _ew(edge_attr, W1[D:])
    s_parts, c_parts = _sc_aggregate(xw, ew, row, col)
    s_parts = s_parts.reshape(NC, NPAD, H)
    c_parts = c_parts.reshape(NC, NPAD)
    out = _final(s_parts, c_parts, W2, b2.reshape(1, O))
    return out[:N]


# speedup vs baseline: 1.5826x; 1.5826x over previous
import jax
import jax.numpy as jnp
from jax import lax
from jax.experimental import pallas as pl
from jax.experimental.pallas import tpu as pltpu
from jax.experimental.pallas import tpu_sc as plsc

N, E, D, DE, H, O = 10000, 320000, 128, 16, 128, 128

NC, NS, L = 2, 16, 16
NW = NC * NS
EPW = E // NW
BLK = 40
NBLK = EPW // BLK
NPAD = 10240
RPT = NPAD // NS


def _xw_body(x_ref, w_ref, b_ref, o_ref):
    o_ref[...] = (
        jnp.dot(x_ref[...], w_ref[...], preferred_element_type=jnp.float32)
        + b_ref[...]
    )


def _xw(x, w1x, b1):
    return pl.pallas_call(
        _xw_body,
        grid=(10,),
        in_specs=[
            pl.BlockSpec((1000, D), lambda i: (i, 0)),
            pl.BlockSpec((D, H), lambda i: (0, 0)),
            pl.BlockSpec((1, H), lambda i: (0, 0)),
        ],
        out_specs=pl.BlockSpec((1000, H), lambda i: (i, 0)),
        out_shape=jax.ShapeDtypeStruct((N, H), jnp.float32),
    )(x, w1x, b1)


def _ew_body(a_ref, w_ref, o_ref):
    o_ref[...] = jnp.dot(a_ref[...], w_ref[...], preferred_element_type=jnp.float32)


def _ew(ea, w1e):
    return pl.pallas_call(
        _ew_body,
        grid=(40,),
        in_specs=[
            pl.BlockSpec((8000, DE), lambda i: (i, 0)),
            pl.BlockSpec((DE, H), lambda i: (0, 0)),
        ],
        out_specs=pl.BlockSpec((8000, H), lambda i: (i, 0)),
        out_shape=jax.ShapeDtypeStruct((E, H), jnp.float32),
    )(ea, w1e)


def _sc_body(xw_hbm, ew_hbm, row_hbm, col_hbm, s_out, c_out,
             rowi_v, coli_v, gath_v, ew_v, ones_v, cb_v, s_sh, cnt_sh, *sems):
    sem_row = sems[0:2]
    sem_col = sems[2:4]
    sem_ew = sems[4:6]
    sem_g = sems[6:8]

    cid = lax.axis_index("c")
    sid = lax.axis_index("s")
    wid = sid * NC + cid

    def _issue_loads(t, b):
        off = wid * EPW + t * BLK
        pltpu.async_copy(row_hbm.at[pl.ds(off, BLK)], rowi_v.at[b], sem_row[b])
        pltpu.async_copy(col_hbm.at[pl.ds(off, BLK)], coli_v.at[b], sem_col[b])
        pltpu.async_copy(ew_hbm.at[pl.ds(off, BLK)], ew_v.at[b], sem_ew[b])

    def _wait_row(b):
        pltpu.make_async_copy(row_hbm.at[pl.ds(0, BLK)], rowi_v.at[b],
                              sem_row[b]).wait()

    def _wait_col_ew(b):
        pltpu.make_async_copy(col_hbm.at[pl.ds(0, BLK)], coli_v.at[b],
                              sem_col[b]).wait()
        pltpu.make_async_copy(ew_hbm.at[pl.ds(0, BLK)], ew_v.at[b],
                              sem_ew[b]).wait()

    def _issue_gather(b):
        pltpu.async_copy(xw_hbm.at[rowi_v.at[b]], gath_v.at[b], sem_g[b])

    def _wait_gather(b):
        pltpu.make_async_copy(xw_hbm.at[rowi_v.at[b]], gath_v.at[b],
                              sem_g[b]).wait()

    def _scatters(b):
        pltpu.sync_copy(gath_v.at[b], s_sh.at[coli_v.at[b]], add=True)
        pltpu.sync_copy(ones_v, cnt_sh.at[coli_v.at[b]], add=True)

    @pl.loop(0, BLK)
    def _fillz(i):
        for j in range(H // L):
            gath_v[0, i, pl.ds(j * L, L)] = jnp.zeros((L,), jnp.float32)

    @pl.loop(0, RPT // L)
    def _fillb(i):
        cb_v[pl.ds(i * L, L)] = jnp.zeros((L,), jnp.float32)

    ones_v[pl.ds(0, L)] = jnp.full((L,), 1.0, jnp.float32)
    ones_v[pl.ds(L, L)] = jnp.full((L,), 1.0, jnp.float32)
    ones_v[pl.ds(BLK - L, L)] = jnp.full((L,), 1.0, jnp.float32)

    @pl.loop(0, RPT // BLK)
    def _zero(t):
        pltpu.sync_copy(gath_v.at[0], s_sh.at[pl.ds(sid * RPT + t * BLK, BLK)])

    pltpu.sync_copy(cb_v, cnt_sh.at[pl.ds(sid * RPT, RPT)])

    plsc.subcore_barrier()

    _issue_loads(0, 0)
    _wait_row(0)
    _issue_gather(0)

    @pl.loop(0, NBLK // 2)
    def _step(s):
        for b in (0, 1):
            t = s * 2 + b
            _wait_col_ew(b)

            @pl.when(t + 1 < NBLK)
            def _():
                _issue_loads(t + 1, b ^ 1)

            _wait_gather(b)

            @pl.when(t + 1 < NBLK)
            def _():
                _wait_row(b ^ 1)
                _issue_gather(b ^ 1)

            @pl.loop(0, BLK)
            def _row(i):
                for j in range(H // L):
                    g = gath_v[b, i, pl.ds(j * L, L)]
                    e = ew_v[b, i, pl.ds(j * L, L)]
                    gath_v[b, i, pl.ds(j * L, L)] = jnp.maximum(g + e, 0.0)

            _scatters(b)

    plsc.subcore_barrier()

    obase = cid * NPAD + sid * RPT

    @pl.loop(0, RPT // BLK)
    def _wout(t):
        pltpu.sync_copy(s_sh.at[pl.ds(sid * RPT + t * BLK, BLK)], gath_v.at[0])
        pltpu.sync_copy(gath_v.at[0], s_out.at[pl.ds(obase + t * BLK, BLK)])

    pltpu.sync_copy(cnt_sh.at[pl.ds(sid * RPT, RPT)], cb_v)
    pltpu.sync_copy(cb_v, c_out.at[pl.ds(obase, RPT)])


def _sc_aggregate(xw, ew, row, col):
    mesh = plsc.VectorSubcoreMesh(core_axis_name="c", subcore_axis_name="s")
    return pl.kernel(
        _sc_body,
        out_type=(
            jax.ShapeDtypeStruct((NC * NPAD, H), jnp.float32),
            jax.ShapeDtypeStruct((NC * NPAD,), jnp.float32),
        ),
        mesh=mesh,
        scratch_types=[
            pltpu.VMEM((2, BLK), jnp.int32),
            pltpu.VMEM((2, BLK), jnp.int32),
            pltpu.VMEM((2, BLK, H), jnp.float32),
            pltpu.VMEM((2, BLK, H), jnp.float32),
            pltpu.VMEM((BLK,), jnp.float32),
            pltpu.VMEM((RPT,), jnp.float32),
            pltpu.VMEM_SHARED((NPAD, H), jnp.float32),
            pltpu.VMEM_SHARED((NPAD,), jnp.float32),
        ] + [pltpu.SemaphoreType.DMA] * 8,
    )(xw, ew, row, col)


def _out_body(s_ref, c_ref, w_ref, b_ref, o_ref):
    s = s_ref[0] + s_ref[1]
    cnt = (c_ref[0] + c_ref[1])[:, None]
    m = jnp.dot(s, w_ref[...], preferred_element_type=jnp.float32)
    o_ref[...] = m / jnp.maximum(cnt, 1.0) + b_ref[...] * (cnt > 0.0)


def _final(s_parts, c_parts, w2, b2):
    return pl.pallas_call(
        _out_body,
        grid=(10,),
        in_specs=[
            pl.BlockSpec((NC, 1024, H), lambda i: (0, i, 0)),
            pl.BlockSpec((NC, 1024), lambda i: (0, i)),
            pl.BlockSpec((H, O), lambda i: (0, 0)),
            pl.BlockSpec((1, O), lambda i: (0, 0)),
        ],
        out_specs=pl.BlockSpec((1024, O), lambda i: (i, 0)),
        out_shape=jax.ShapeDtypeStruct((NPAD, O), jnp.float32),
    )(s_parts, c_parts, w2, b2)


def kernel(x, edge_index, edge_attr, W1, b1, W2, b2):
    row = edge_index[0]
    col = edge_index[1]
    xw = _xw(x, W1[:D], b1.reshape(1, H))
    ew = _ew(edge_attr, W1[D:])
    s_parts, c_parts = _sc_aggregate(xw, ew, row, col)
    s_parts = s_parts.reshape(NC, NPAD, H)
    c_parts = c_parts.reshape(NC, NPAD)
    out = _final(s_parts, c_parts, W2, b2.reshape(1, O))
    return out[:N]
